# Initial kernel scaffold; baseline (speedup 1.0000x reference)
import functools

import jax
import jax.numpy as jnp
from jax import lax
from jax.experimental import pallas as pl
from jax.experimental.pallas import tpu as pltpu
from jax.experimental.pallas import tpu_sc as plsc

B, C, T = 16, 9, 4096
L = 16
NVEC = T // L
PBUF = T + L
SBUF = T + L

_mesh = plsc.VectorSubcoreMesh(core_axis_name="c", subcore_axis_name="s")


@functools.partial(
    pl.kernel,
    mesh=_mesh,
    out_type=jax.ShapeDtypeStruct((B, 2, T), jnp.float32),
    scratch_types=[
        pltpu.VMEM((PBUF,), jnp.float32),
        pltpu.VMEM((PBUF,), jnp.float32),
        pltpu.VMEM((SBUF,), jnp.float32),
        pltpu.VMEM((T,), jnp.float32),
        pltpu.SemaphoreType.DMA,
        pltpu.SemaphoreType.DMA,
    ],
)
def _finger_mask_sc(gl_hbm, out_hbm, pbuf, rbuf, sbuf, obuf, sem_p, sem_r):
    wid = lax.axis_index("s") * 2 + lax.axis_index("c")
    b = wid // 2
    f = wid % 2

    zeros = jnp.zeros((L,), jnp.float32)
    pbuf[pl.ds(0, L)] = zeros
    rbuf[pl.ds(0, L)] = zeros
    sbuf[pl.ds(0, L)] = zeros
    sbuf[pl.ds(T, L)] = zeros

    cp_p = pltpu.async_copy(gl_hbm.at[b, 2 * f], pbuf.at[pl.ds(8, T)], sem_p)
    cp_r = pltpu.async_copy(gl_hbm.at[b, 2 * f + 1], rbuf.at[pl.ds(8, T)], sem_r)
    cp_p.wait()
    cp_r.wait()

    iota2 = lax.iota(jnp.int32, L) * 2
    false_v = jnp.zeros((L,), jnp.bool_)

    def pass1(i, carry):
        s_carry, accp, accr = carry
        off = i * L
        x = pbuf[pl.ds(off + 8, L)]
        xm = pbuf[pl.ds(off + 7, L)]
        y = rbuf[pl.ds(off + 8, L)]
        ym = rbuf[pl.ds(off + 7, L)]
        p_on = (x - xm) > 0
        r_on = (y - ym) > 0
        t2 = iota2 + (off * 2)
        e = jnp.where(p_on, t2 + 1, jnp.where(r_on, t2, -1))
        ec = plsc.cummax(e)
        s_bit = jnp.where(ec >= 0, lax.rem(ec, 2), s_carry)
        sbuf[pl.ds(off + 4, L)] = s_bit.astype(jnp.float32)
        last = jnp.max(ec)
        s_carry = jnp.where(last >= 0, lax.rem(last, 2), s_carry)
        return s_carry, accp | p_on, accr | r_on

    s_last, accp, accr = lax.fori_loop(
        0, NVEC, pass1, (jnp.int32(0), false_v, false_v))
    valid = jnp.any(accp) & jnp.any(accr)
    vf = jnp.where(valid, jnp.float32(1.0), jnp.float32(0.0))

    def pass2(i, _):
        off = i * L
        m = sbuf[pl.ds(off, L)]
        for k in range(1, 8):
            m = jnp.maximum(m, sbuf[pl.ds(off + k, L)])
        obuf[pl.ds(off, L)] = m * vf
        return 0

    lax.fori_loop(0, NVEC, pass2, 0)
    pltpu.sync_copy(obuf, out_hbm.at[b, f])


def kernel(gesture_labels):
    return _finger_mask_sc(gesture_labels)


# speedup vs baseline: 280.6643x; 280.6643x over previous
import functools

import jax
import jax.numpy as jnp
from jax import lax
from jax.experimental import pallas as pl
from jax.experimental.pallas import tpu as pltpu
from jax.experimental.pallas import tpu_sc as plsc

B, C, T = 16, 9, 4096
L = 16
NVEC = T // L
PBUF = T + L
SBUF = T + L

_mesh = plsc.VectorSubcoreMesh(core_axis_name="c", subcore_axis_name="s")


@functools.partial(
    pl.kernel,
    mesh=_mesh,
    out_type=jax.ShapeDtypeStruct((B * 2 * T,), jnp.float32),
    compiler_params=pltpu.CompilerParams(needs_layout_passes=False),
    scratch_types=[
        pltpu.VMEM((PBUF,), jnp.float32),
        pltpu.VMEM((PBUF,), jnp.float32),
        pltpu.VMEM((SBUF,), jnp.float32),
        pltpu.VMEM((T,), jnp.float32),
        pltpu.SemaphoreType.DMA,
        pltpu.SemaphoreType.DMA,
    ],
)
def _finger_mask_sc(gl_hbm, out_hbm, pbuf, rbuf, sbuf, obuf, sem_p, sem_r):
    wid = lax.axis_index("s") * 2 + lax.axis_index("c")

    zeros = jnp.zeros((L,), jnp.float32)
    pbuf[pl.ds(0, L)] = zeros
    rbuf[pl.ds(0, L)] = zeros
    sbuf[pl.ds(0, L)] = zeros
    sbuf[pl.ds(T, L)] = zeros

    base = wid * (2 * T)
    cp_p = pltpu.async_copy(gl_hbm.at[pl.ds(base, T)], pbuf.at[pl.ds(8, T)], sem_p)
    cp_r = pltpu.async_copy(gl_hbm.at[pl.ds(base + T, T)], rbuf.at[pl.ds(8, T)], sem_r)
    cp_p.wait()
    cp_r.wait()

    iota2 = lax.iota(jnp.int32, L) * 2
    false_v = jnp.zeros((L,), jnp.bool_)

    def pass1(i, carry):
        s_carry, accp, accr = carry
        off = i * L
        x = pbuf[pl.ds(off + 8, L)]
        xm = pbuf[pl.ds(off + 7, L)]
        y = rbuf[pl.ds(off + 8, L)]
        ym = rbuf[pl.ds(off + 7, L)]
        p_on = (x - xm) > 0
        r_on = (y - ym) > 0
        t2 = iota2 + (off * 2)
        e = jnp.where(p_on, t2 + 1, jnp.where(r_on, t2, -1))
        ec = plsc.cummax(e)
        s_bit = jnp.where(ec >= 0, lax.rem(ec, 2), s_carry)
        sbuf[pl.ds(off + 4, L)] = s_bit.astype(jnp.float32)
        last = jnp.max(ec)
        s_carry = jnp.where(last >= 0, lax.rem(last, 2), s_carry)
        return s_carry, accp | p_on, accr | r_on

    s_last, accp, accr = lax.fori_loop(
        0, NVEC, pass1, (jnp.int32(0), false_v, false_v))
    valid = jnp.any(accp) & jnp.any(accr)
    vf = jnp.where(valid, jnp.float32(1.0), jnp.float32(0.0))

    def pass2(i, _):
        off = i * L
        m = sbuf[pl.ds(off, L)]
        for k in range(1, 8):
            m = jnp.maximum(m, sbuf[pl.ds(off + k, L)])
        obuf[pl.ds(off, L)] = m * vf
        return 0

    lax.fori_loop(0, NVEC, pass2, 0)
    pltpu.sync_copy(obuf, out_hbm.at[pl.ds(wid * T, T)])


def kernel(gesture_labels):
    gl4 = gesture_labels[:, :4, :].reshape(-1)
    out = _finger_mask_sc(gl4)
    return out.reshape(B, 2, T)
